# Initial kernel scaffold; baseline (speedup 1.0000x reference)
import jax
import jax.numpy as jnp
from jax.experimental import pallas as pl


def kernel(cd_p, css_matrix, dss_matrix, Wrdx, brdx, Wrdy, brdy, Wx1, bx1, Wx2, bx2, Wy1, by1, Wy2, by2, msp_w1, msp_b1, msp_w3, msp_b3, msp_gamma, msp_beta, Wcx, bcx, Wcy, bcy):
    raise NotImplementedError("write your pallas kernel here")


# speedup vs baseline: 2.7562x; 2.7562x over previous
import functools

import jax
import jax.numpy as jnp
from jax.experimental import pallas as pl

F32 = jnp.float32
K_NN = 16


def _mm_kernel(a_ref, b_ref, bias_ref, o_ref, *, act, contract):
    dn = (((contract[0],), (contract[1],)), ((), ()))
    acc = jax.lax.dot_general(a_ref[...], b_ref[...], dn,
                              preferred_element_type=F32)
    acc = acc + bias_ref[...]
    if act == "sigmoid":
        acc = jax.nn.sigmoid(acc)
    o_ref[...] = acc


def _mm(a, b, bias, act=None, contract=(1, 1)):
    m = a.shape[1 - contract[0]]
    n = b.shape[1 - contract[1]]
    return pl.pallas_call(
        functools.partial(_mm_kernel, act=act, contract=contract),
        out_shape=jax.ShapeDtypeStruct((m, n), F32),
    )(a, b, bias)


def _mm_kchunk(a, b, contract, kb=256):
    ka = a.shape[contract[0]]
    m = a.shape[1 - contract[0]]
    n = b.shape[1 - contract[1]]
    dn = (((contract[0],), (contract[1],)), ((), ()))

    def kern(a_ref, b_ref, o_ref):
        k = pl.program_id(0)
        part = jax.lax.dot_general(a_ref[...], b_ref[...], dn,
                                   preferred_element_type=F32)

        @pl.when(k == 0)
        def _():
            o_ref[...] = part

        @pl.when(k > 0)
        def _():
            o_ref[...] = o_ref[...] + part

    aspec = (pl.BlockSpec((m, kb), lambda k: (0, k)) if contract[0] == 1
             else pl.BlockSpec((kb, m), lambda k: (k, 0)))
    bspec = (pl.BlockSpec((n, kb), lambda k: (0, k)) if contract[1] == 1
             else pl.BlockSpec((kb, n), lambda k: (k, 0)))
    return pl.pallas_call(
        kern, grid=(ka // kb,),
        in_specs=[aspec, bspec],
        out_specs=pl.BlockSpec((m, n), lambda k: (0, 0)),
        out_shape=jax.ShapeDtypeStruct((m, n), F32))(a, b)


def _dist_kernel(xa_ref, xb_ref, sqr_ref, sqc_ref, d_ref):
    xy = jax.lax.dot_general(xb_ref[...], xa_ref[...],
                             (((1,), (1,)), ((), ())),
                             preferred_element_type=F32)
    d_ref[...] = jnp.maximum(sqc_ref[...] + sqr_ref[...] - 2.0 * xy, 0.0)


def _dist(x, sq_row, sq_col, r=256):
    n, f = x.shape
    return pl.pallas_call(
        _dist_kernel,
        grid=(n // r,),
        in_specs=[
            pl.BlockSpec((n, f), lambda i: (0, 0)),
            pl.BlockSpec((r, f), lambda i: (i, 0)),
            pl.BlockSpec((1, n), lambda i: (0, 0)),
            pl.BlockSpec((r, 1), lambda i: (i, 0)),
        ],
        out_specs=pl.BlockSpec((r, n), lambda i: (i, 0)),
        out_shape=jax.ShapeDtypeStruct((n, n), F32),
    )(x, x, sq_row, sq_col)


def _select_kernel(d_ref, m_ref, *, n, r):
    d = d_ref[...]
    iota = jax.lax.broadcasted_iota(jnp.int32, (r, n), 1)
    neg_inf = jnp.float32(-jnp.inf)

    def body(_, carry):
        work, macc = carry
        mv = jnp.min(work, axis=1, keepdims=True)
        idx = jnp.min(jnp.where(work == mv, iota, n), axis=1, keepdims=True)
        oh = iota == idx
        macc = jnp.where(oh, -mv, macc)
        work = jnp.where(oh, jnp.float32(jnp.inf), work)
        return work, macc

    _, macc = jax.lax.fori_loop(
        0, K_NN, body, (d, jnp.full((r, n), neg_inf, F32)))
    m_ref[...] = macc


def _select(d, r=256):
    n = d.shape[0]
    return pl.pallas_call(
        functools.partial(_select_kernel, n=n, r=r),
        grid=(n // r,),
        in_specs=[pl.BlockSpec((r, n), lambda i: (i, 0))],
        out_specs=pl.BlockSpec((r, n), lambda i: (i, 0)),
        out_shape=jax.ShapeDtypeStruct((n, n), F32),
    )(d)


def _knn_adj(x):
    n = x.shape[0]
    sq = jnp.sum(x * x, axis=1)
    d = _dist(x, sq.reshape(1, n), sq.reshape(n, 1))
    avg = jnp.mean(d, axis=1, keepdims=True) + 1e-8
    negd = _select(d)
    at = jnp.exp(negd / avg) + jnp.eye(n, dtype=d.dtype)
    deg = jnp.sum(at, axis=0)
    dinv = jnp.where(deg > 0, 1.0 / jnp.sqrt(deg), 0.0)
    return at, dinv.reshape(n, 1)


def _gcn_kernel(at_ref, dinv_ref, x_ref, w_ref, b_ref, dinv_all_ref, o_ref):
    k = pl.program_id(0)
    xw = jax.lax.dot_general(x_ref[...], w_ref[...], (((1,), (1,)), ((), ())),
                             preferred_element_type=F32)
    v = dinv_ref[...] * xw
    part = jax.lax.dot_general(at_ref[...], v, (((0,), (0,)), ((), ())),
                               preferred_element_type=F32)

    @pl.when(k == 0)
    def _():
        o_ref[...] = part

    @pl.when(k > 0)
    def _():
        o_ref[...] = o_ref[...] + part

    @pl.when(k == pl.num_programs(0) - 1)
    def _():
        o_ref[...] = jnp.maximum(
            dinv_all_ref[...] * o_ref[...] + b_ref[...], 0.0)


def _gcn(at, dinv, x, w, b_row, kb=None):
    n, feat = x.shape
    kb = n if kb is None else kb
    return pl.pallas_call(
        _gcn_kernel,
        grid=(n // kb,),
        in_specs=[
            pl.BlockSpec((kb, n), lambda k: (k, 0)),
            pl.BlockSpec((kb, 1), lambda k: (k, 0)),
            pl.BlockSpec((kb, feat), lambda k: (k, 0)),
            pl.BlockSpec((feat, feat), lambda k: (0, 0)),
            pl.BlockSpec((1, feat), lambda k: (0, 0)),
            pl.BlockSpec((n, 1), lambda k: (0, 0)),
        ],
        out_specs=pl.BlockSpec((n, feat), lambda k: (0, 0)),
        out_shape=jax.ShapeDtypeStruct((n, feat), F32),
    )(at, dinv, x, w, b_row, dinv)


def _msp_kernel(f_ref, p_ref, o_ref, *, n, w):
    g = f_ref[0]
    p = p_ref[...]
    w1, b1, b3, gamma, beta = p[0, 0], p[0, 1], p[0, 2], p[0, 3], p[0, 4]
    rm = jnp.mean(g, axis=1, keepdims=True)
    cm = jnp.mean(g, axis=0, keepdims=True)
    pre = g * jax.nn.sigmoid(w1 * rm + b1) * jax.nn.sigmoid(w1 * cm + b1)
    mu = jnp.mean(pre)
    var = jnp.mean((pre - mu) ** 2)
    x1 = (pre - mu) * jax.lax.rsqrt(var + 1e-5) * gamma + beta
    pad = jax.lax.pad(g, jnp.float32(0.0), ((1, 1, 0), (1, 1, 0)))
    x2 = jnp.zeros((n, w), F32)
    for u in range(3):
        for v in range(3):
            x2 = x2 + p[0, 5 + 3 * u + v] * jax.lax.slice(
                pad, (u, v), (u + n, v + w))
    x2 = x2 + b3
    o_ref[0] = g * jax.nn.sigmoid(x1 + x2)


def _msp(feats, pvec):
    gnum, n, w = feats.shape
    return pl.pallas_call(
        functools.partial(_msp_kernel, n=n, w=w),
        grid=(gnum,),
        in_specs=[
            pl.BlockSpec((1, n, w), lambda i: (i, 0, 0)),
            pl.BlockSpec((1, 14), lambda i: (0, 0)),
        ],
        out_specs=pl.BlockSpec((1, n, w), lambda i: (i, 0, 0)),
        out_shape=jax.ShapeDtypeStruct((gnum, n, w), F32),
    )(feats, pvec)


def _branch_feats(x0, feat0, w1, b1r, w2, b2r):
    feats = [x0]
    a, dinv = _knn_adj(feat0)
    x = _gcn(a, dinv, x0, w1, b1r)
    feats.append(x)
    for _ in range(3):
        a, dinv = _knn_adj(x)
        x = _gcn(a, dinv, x, w2, b2r)
        feats.append(x)
    return jnp.stack(feats, axis=0)


def kernel(cd_p, css_matrix, dss_matrix, Wrdx, brdx, Wrdy, brdy, Wx1, bx1,
           Wx2, bx2, Wy1, by1, Wy2, by2, msp_w1, msp_b1, msp_w3, msp_b3,
           msp_gamma, msp_beta, Wcx, bcx, Wcy, bcy):
    nc, nd = cd_p.shape
    feat = Wrdx.shape[0]

    x_c = _mm_kchunk(cd_p, Wrdx, (1, 1)) + brdx
    x_d = _mm_kchunk(cd_p, Wrdy, (0, 1)) + brdy

    c_f = _branch_feats(x_c, css_matrix, Wx1, bx1.reshape(1, -1),
                        Wx2, bx2.reshape(1, -1))
    d_f = _branch_feats(x_d, dss_matrix, Wy1, by1.reshape(1, -1),
                        Wy2, by2.reshape(1, -1))

    pvec = jnp.concatenate(
        [msp_w1, msp_b1, msp_b3, msp_gamma, msp_beta,
         msp_w3.reshape(-1)]).reshape(1, 14).astype(F32)
    out_c = _msp(c_f, pvec)
    out_d = _msp(d_f, pvec)

    flat_c = jnp.transpose(out_c, (1, 0, 2)).reshape(nc, 5 * feat)
    flat_d = jnp.transpose(out_d, (1, 0, 2)).reshape(nd, 5 * feat)
    x_feat = _mm(flat_c, Wcx.reshape(-1, 5 * feat), bcx.reshape(1, -1))
    y_feat = _mm(flat_d, Wcy.reshape(-1, 5 * feat), bcy.reshape(1, -1))

    zeros_bias = jnp.zeros((1, nd), F32)
    return _mm(x_feat, y_feat, zeros_bias, act="sigmoid")


# speedup vs baseline: 2.7840x; 1.0101x over previous
import functools

import jax
import jax.numpy as jnp
from jax.experimental import pallas as pl

F32 = jnp.float32
K_NN = 16


def _mm_kernel(a_ref, b_ref, bias_ref, o_ref, *, act, contract):
    dn = (((contract[0],), (contract[1],)), ((), ()))
    acc = jax.lax.dot_general(a_ref[...], b_ref[...], dn,
                              preferred_element_type=F32)
    acc = acc + bias_ref[...]
    if act == "sigmoid":
        acc = jax.nn.sigmoid(acc)
    o_ref[...] = acc


def _mm(a, b, bias, act=None, contract=(1, 1)):
    m = a.shape[1 - contract[0]]
    n = b.shape[1 - contract[1]]
    return pl.pallas_call(
        functools.partial(_mm_kernel, act=act, contract=contract),
        out_shape=jax.ShapeDtypeStruct((m, n), F32),
    )(a, b, bias)


def _mm_kchunk(a, b, contract, kb=256):
    ka = a.shape[contract[0]]
    m = a.shape[1 - contract[0]]
    n = b.shape[1 - contract[1]]
    dn = (((contract[0],), (contract[1],)), ((), ()))

    def kern(a_ref, b_ref, o_ref):
        k = pl.program_id(0)
        part = jax.lax.dot_general(a_ref[...], b_ref[...], dn,
                                   preferred_element_type=F32)

        @pl.when(k == 0)
        def _():
            o_ref[...] = part

        @pl.when(k > 0)
        def _():
            o_ref[...] = o_ref[...] + part

    aspec = (pl.BlockSpec((m, kb), lambda k: (0, k)) if contract[0] == 1
             else pl.BlockSpec((kb, m), lambda k: (k, 0)))
    bspec = (pl.BlockSpec((n, kb), lambda k: (0, k)) if contract[1] == 1
             else pl.BlockSpec((kb, n), lambda k: (k, 0)))
    return pl.pallas_call(
        kern, grid=(ka // kb,),
        in_specs=[aspec, bspec],
        out_specs=pl.BlockSpec((m, n), lambda k: (0, 0)),
        out_shape=jax.ShapeDtypeStruct((m, n), F32))(a, b)


def _dist_kernel(xa_ref, xb_ref, sqr_ref, sqc_ref, d_ref, avg_ref, *, n):
    xy = jax.lax.dot_general(xb_ref[...], xa_ref[...],
                             (((1,), (1,)), ((), ())),
                             preferred_element_type=F32)
    dblk = jnp.maximum(sqc_ref[...] + sqr_ref[...] - 2.0 * xy, 0.0)
    d_ref[...] = dblk
    avg_ref[...] = jnp.sum(dblk, axis=1, keepdims=True) * F32(1.0 / n) + 1e-8


def _dist(x, sq_row, sq_col, r=256):
    n, f = x.shape
    return pl.pallas_call(
        functools.partial(_dist_kernel, n=n),
        grid=(n // r,),
        in_specs=[
            pl.BlockSpec((n, f), lambda i: (0, 0)),
            pl.BlockSpec((r, f), lambda i: (i, 0)),
            pl.BlockSpec((1, n), lambda i: (0, 0)),
            pl.BlockSpec((r, 1), lambda i: (i, 0)),
        ],
        out_specs=[pl.BlockSpec((r, n), lambda i: (i, 0)),
                   pl.BlockSpec((r, 1), lambda i: (i, 0))],
        out_shape=[jax.ShapeDtypeStruct((n, n), F32),
                   jax.ShapeDtypeStruct((n, 1), F32)],
    )(x, x, sq_row, sq_col)


def _select_kernel(d_ref, m_ref, *, n, r):
    d = d_ref[...]
    iota = jax.lax.broadcasted_iota(jnp.int32, (r, n), 1)
    neg_inf = jnp.float32(-jnp.inf)

    def body(_, carry):
        work, macc = carry
        mv = jnp.min(work, axis=1, keepdims=True)
        idx = jnp.min(jnp.where(work == mv, iota, n), axis=1, keepdims=True)
        oh = iota == idx
        macc = jnp.where(oh, -mv, macc)
        work = jnp.where(oh, jnp.float32(jnp.inf), work)
        return work, macc

    _, macc = jax.lax.fori_loop(
        0, K_NN, body, (d, jnp.full((r, n), neg_inf, F32)))
    m_ref[...] = macc


def _select(d, r=256):
    n = d.shape[0]
    return pl.pallas_call(
        functools.partial(_select_kernel, n=n, r=r),
        grid=(n // r,),
        in_specs=[pl.BlockSpec((r, n), lambda i: (i, 0))],
        out_specs=pl.BlockSpec((r, n), lambda i: (i, 0)),
        out_shape=jax.ShapeDtypeStruct((n, n), F32),
    )(d)


def _knn_adj(x):
    n = x.shape[0]
    sq = jnp.sum(x * x, axis=1)
    d, avg = _dist(x, sq.reshape(1, n), sq.reshape(n, 1))
    negd = _select(d)
    at = jnp.exp(negd / avg) + jnp.eye(n, dtype=d.dtype)
    deg = jnp.sum(at, axis=0)
    dinv = jnp.where(deg > 0, 1.0 / jnp.sqrt(deg), 0.0)
    return at, dinv.reshape(n, 1)


def _gcn_kernel(at_ref, dinv_ref, x_ref, w_ref, b_ref, dinv_all_ref, o_ref):
    k = pl.program_id(0)
    xw = jax.lax.dot_general(x_ref[...], w_ref[...], (((1,), (1,)), ((), ())),
                             preferred_element_type=F32)
    v = dinv_ref[...] * xw
    part = jax.lax.dot_general(at_ref[...], v, (((0,), (0,)), ((), ())),
                               preferred_element_type=F32)

    @pl.when(k == 0)
    def _():
        o_ref[...] = part

    @pl.when(k > 0)
    def _():
        o_ref[...] = o_ref[...] + part

    @pl.when(k == pl.num_programs(0) - 1)
    def _():
        o_ref[...] = jnp.maximum(
            dinv_all_ref[...] * o_ref[...] + b_ref[...], 0.0)


def _gcn(at, dinv, x, w, b_row, kb=None):
    n, feat = x.shape
    kb = n if kb is None else kb
    return pl.pallas_call(
        _gcn_kernel,
        grid=(n // kb,),
        in_specs=[
            pl.BlockSpec((kb, n), lambda k: (k, 0)),
            pl.BlockSpec((kb, 1), lambda k: (k, 0)),
            pl.BlockSpec((kb, feat), lambda k: (k, 0)),
            pl.BlockSpec((feat, feat), lambda k: (0, 0)),
            pl.BlockSpec((1, feat), lambda k: (0, 0)),
            pl.BlockSpec((n, 1), lambda k: (0, 0)),
        ],
        out_specs=pl.BlockSpec((n, feat), lambda k: (0, 0)),
        out_shape=jax.ShapeDtypeStruct((n, feat), F32),
    )(at, dinv, x, w, b_row, dinv)


def _msp_kernel(f_ref, p_ref, o_ref, *, n, w):
    g = f_ref[0]
    p = p_ref[...]
    w1, b1, b3, gamma, beta = p[0, 0], p[0, 1], p[0, 2], p[0, 3], p[0, 4]
    rm = jnp.mean(g, axis=1, keepdims=True)
    cm = jnp.mean(g, axis=0, keepdims=True)
    pre = g * jax.nn.sigmoid(w1 * rm + b1) * jax.nn.sigmoid(w1 * cm + b1)
    mu = jnp.mean(pre)
    var = jnp.mean((pre - mu) ** 2)
    x1 = (pre - mu) * jax.lax.rsqrt(var + 1e-5) * gamma + beta
    pad = jax.lax.pad(g, jnp.float32(0.0), ((1, 1, 0), (1, 1, 0)))
    x2 = jnp.zeros((n, w), F32)
    for u in range(3):
        for v in range(3):
            x2 = x2 + p[0, 5 + 3 * u + v] * jax.lax.slice(
                pad, (u, v), (u + n, v + w))
    x2 = x2 + b3
    o_ref[0] = g * jax.nn.sigmoid(x1 + x2)


def _msp(feats, pvec):
    gnum, n, w = feats.shape
    return pl.pallas_call(
        functools.partial(_msp_kernel, n=n, w=w),
        grid=(gnum,),
        in_specs=[
            pl.BlockSpec((1, n, w), lambda i: (i, 0, 0)),
            pl.BlockSpec((1, 14), lambda i: (0, 0)),
        ],
        out_specs=pl.BlockSpec((1, n, w), lambda i: (i, 0, 0)),
        out_shape=jax.ShapeDtypeStruct((gnum, n, w), F32),
    )(feats, pvec)


def _branch_feats(x0, feat0, w1, b1r, w2, b2r):
    feats = [x0]
    a, dinv = _knn_adj(feat0)
    x = _gcn(a, dinv, x0, w1, b1r)
    feats.append(x)
    for _ in range(3):
        a, dinv = _knn_adj(x)
        x = _gcn(a, dinv, x, w2, b2r)
        feats.append(x)
    return jnp.stack(feats, axis=0)


def kernel(cd_p, css_matrix, dss_matrix, Wrdx, brdx, Wrdy, brdy, Wx1, bx1,
           Wx2, bx2, Wy1, by1, Wy2, by2, msp_w1, msp_b1, msp_w3, msp_b3,
           msp_gamma, msp_beta, Wcx, bcx, Wcy, bcy):
    nc, nd = cd_p.shape
    feat = Wrdx.shape[0]

    x_c = _mm_kchunk(cd_p, Wrdx, (1, 1)) + brdx
    x_d = _mm_kchunk(cd_p, Wrdy, (0, 1)) + brdy

    c_f = _branch_feats(x_c, css_matrix, Wx1, bx1.reshape(1, -1),
                        Wx2, bx2.reshape(1, -1))
    d_f = _branch_feats(x_d, dss_matrix, Wy1, by1.reshape(1, -1),
                        Wy2, by2.reshape(1, -1))

    pvec = jnp.concatenate(
        [msp_w1, msp_b1, msp_b3, msp_gamma, msp_beta,
         msp_w3.reshape(-1)]).reshape(1, 14).astype(F32)
    out_c = _msp(c_f, pvec)
    out_d = _msp(d_f, pvec)

    flat_c = jnp.transpose(out_c, (1, 0, 2)).reshape(nc, 5 * feat)
    flat_d = jnp.transpose(out_d, (1, 0, 2)).reshape(nd, 5 * feat)
    x_feat = _mm(flat_c, Wcx.reshape(-1, 5 * feat), bcx.reshape(1, -1))
    y_feat = _mm(flat_d, Wcy.reshape(-1, 5 * feat), bcy.reshape(1, -1))

    zeros_bias = jnp.zeros((1, nd), F32)
    return _mm(x_feat, y_feat, zeros_bias, act="sigmoid")
